# baseline (speedup 1.0000x reference)
import functools

import jax
import jax.numpy as jnp
from jax import lax
from jax.experimental import pallas as pl
from jax.experimental.pallas import tpu as pltpu
from jax.experimental.pallas import tpu_sc as plsc

_VOCAB = 100000
_DIM = 128
_BATCH = 4096
_L = 17
_LSC = 6
_LP = 16
_C = 16
_NC = 2
_NS = 16
_NW = _NC * _NS
_RW = _BATCH // _NW
_NCH = _RW // _C

_WIDE = ((7, 780, 7), (8, 389, 4), (9, 194, 2),
         (10, 96, 1))


def _sc_body(input_hbm, target_hbm, table_hbm, out_hbm,
             tgt_v, idx8_v, w_v, e_v, dots_v,
             sem_in0, sem_in1, sem_g0, sem_g1, sem_o0, sem_o1):
    sem_in = (sem_in0, sem_in1)
    sem_g = (sem_g0, sem_g1)
    sem_o = (sem_o0, sem_o1)
    wid = lax.axis_index("s") * _NC + lax.axis_index("c")
    lanes = lax.iota(jnp.int32, 16)

    def issue_in(ch):
        p = ch & 1
        base = wid * _RW + ch * _C
        return [
            pltpu.async_copy(target_hbm.at[pl.ds(base, _C)],
                             tgt_v.at[p], sem_in[p]),
            pltpu.async_copy(input_hbm.at[pl.ds(base * _DIM, _C * _DIM)],
                             w_v.at[p], sem_in[p]),
        ]

    def idx_and_gather(ch):
        p = ch & 1
        m = tgt_v[p] + _VOCAB
        for kk in range(1, _LSC + 1):
            idx8_v[p, pl.ds((kk - 1) * 16, 16)] = jnp.right_shift(m, kk) - 1
        return [
            pltpu.async_copy(table_hbm.at[idx8_v.at[p, pl.ds(0, _LSC * _C)]],
                             e_v.at[p, pl.ds(0, _LSC * _C)], sem_g[p]),
        ]

    def compute_rows(ch):
        p = ch & 1

        def row_body(b, carry):
            wb = [w_v[p, pl.ds(b * _DIM + c * 16, 16)] for c in range(8)]
            dots0 = jnp.zeros((16,), jnp.float32)
            for kk in range(_LSC):
                row = kk * _C + b
                acc = e_v[p, row, pl.ds(0, 16)] * wb[0]
                for c in range(1, 8):
                    acc = acc + e_v[p, row, pl.ds(c * 16, 16)] * wb[c]
                for s in (1, 2, 4, 8):
                    acc = acc + acc.at[lanes ^ s].get(
                        mode="promise_in_bounds")
                dots0 = jnp.where(lanes == kk, acc, dots0)
            dots_v[p, pl.ds(b * _LP, 16)] = dots0
            return carry

        lax.fori_loop(0, _C, row_body, 0)

    def issue_out(ch):
        p = ch & 1
        base = wid * _RW + ch * _C
        return pltpu.async_copy(dots_v.at[p],
                                out_hbm.at[pl.ds(base * _LP, _C * _LP)],
                                sem_o[p])

    in_c = {0: issue_in(0), 1: issue_in(1)}
    for c in in_c[0]:
        c.wait()
    g_c = {0: idx_and_gather(0)}
    out_c = {}
    for ch in range(_NCH):
        if ch + 1 < _NCH:
            for c in in_c[ch + 1]:
                c.wait()
            g_c[ch + 1] = idx_and_gather(ch + 1)
        for c in g_c[ch]:
            c.wait()
        if ch >= 2:
            out_c[ch - 2].wait()
        compute_rows(ch)
        out_c[ch] = issue_out(ch)
        if ch + 2 < _NCH:
            in_c[ch + 2] = issue_in(ch + 2)
    out_c[_NCH - 2].wait()
    out_c[_NCH - 1].wait()


_sc_dots = functools.partial(
    pl.kernel,
    mesh=plsc.VectorSubcoreMesh(core_axis_name="c", subcore_axis_name="s"),
    out_type=jax.ShapeDtypeStruct((_BATCH * _LP,), jnp.float32),
    scratch_types=[
        pltpu.VMEM((2, _C), jnp.int32),
        pltpu.VMEM((2, 128), jnp.int32),
        pltpu.VMEM((2, _C * _DIM), jnp.float32),
        pltpu.VMEM((2, _LSC * _C, _DIM), jnp.float32),
        pltpu.VMEM((2, _C * _LP), jnp.float32),
        pltpu.SemaphoreType.DMA,
        pltpu.SemaphoreType.DMA,
        pltpu.SemaphoreType.DMA,
        pltpu.SemaphoreType.DMA,
        pltpu.SemaphoreType.DMA,
        pltpu.SemaphoreType.DMA,
    ],
)(_sc_body)


def _tc_top_body(x_ref, tgt_ref, tab_ref, out_ref):
    x = x_ref[...]
    m = tgt_ref[...] + _VOCAB
    dims = (((1,), (1,)), ((), ()))

    def score(base):
        t = tab_ref[pl.ds(base, 128), :]
        return lax.dot_general(x, t, dims,
                               preferred_element_type=jnp.float32)

    s_lo = score(0)
    kcol = 10 + lax.broadcasted_iota(jnp.int32, (_BATCH, 8), 1)
    mk = jnp.right_shift(m, kcol)
    off_lo = jnp.where((kcol > 10) & (mk >= 1), mk - 1, 0)
    top = jnp.take_along_axis(s_lo, off_lo, axis=1)
    turn = jnp.where((jnp.right_shift(m, kcol - 1) & 1) == 0, 1.0, -1.0)
    coef = jnp.where(mk >= 1, turn, 0.0)
    total = jnp.sum(jax.nn.log_sigmoid(top * coef) *
                    jnp.where(kcol > 10, 1.0, 0.0))
    for k, base, nslices in _WIDE:
        off = jnp.right_shift(m, k) - 1 - base
        sel = None
        for i in range(nslices):
            s_i = score(base + 128 * i)
            c_i = jnp.take_along_axis(
                s_i, jnp.clip(off - 128 * i, 0, 127), axis=1)
            sel = c_i if sel is None else jnp.where(off < 128 * i, sel, c_i)
        turn_k = jnp.where((jnp.right_shift(m, k - 1) & 1) == 0, 1.0, -1.0)
        total = total + jnp.sum(jax.nn.log_sigmoid(sel * turn_k))
    out_ref[0, 0] = total


def _tc_top(input_word, tgt2d, table):
    return pl.pallas_call(
        _tc_top_body,
        grid=(1,),
        in_specs=[
            pl.BlockSpec((_BATCH, _DIM), lambda i: (0, 0)),
            pl.BlockSpec((_BATCH, 1), lambda i: (0, 0)),
            pl.BlockSpec((2048, _DIM), lambda i: (0, 0)),
        ],
        out_shape=jax.ShapeDtypeStruct((1, 1), jnp.float32),
        out_specs=pl.BlockSpec(memory_space=pltpu.SMEM),
    )(input_word, tgt2d, table)


def _tc_loss_body(dots_ref, tgt_ref, top_ref, out_ref):
    m = tgt_ref[...] + _VOCAB
    col = lax.broadcasted_iota(jnp.int32, (_BATCH, _LP), 1)
    turn = jnp.where((jnp.right_shift(m, col) & 1) == 0, 1.0, -1.0)
    x = dots_ref[...] * turn
    ls = jnp.where(col < _LSC, jax.nn.log_sigmoid(x), 0.0)
    out_ref[0, 0] = -(jnp.sum(ls) + top_ref[0, 0]) / _BATCH


def _tc_loss(dots2d, tgt2d, topsum):
    return pl.pallas_call(
        _tc_loss_body,
        in_specs=[
            pl.BlockSpec((_BATCH, _LP), lambda: (0, 0)),
            pl.BlockSpec((_BATCH, 1), lambda: (0, 0)),
            pl.BlockSpec(memory_space=pltpu.SMEM),
        ],
        out_shape=jax.ShapeDtypeStruct((1, 1), jnp.float32),
        out_specs=pl.BlockSpec(memory_space=pltpu.SMEM),
    )(dots2d, tgt2d, topsum)


def kernel(input_word, target, output_matrix):
    dots_flat = _sc_dots(input_word.reshape(-1), target, output_matrix)
    topsum = _tc_top(input_word, target.reshape(_BATCH, 1), output_matrix)
    loss = _tc_loss(dots_flat.reshape(_BATCH, _LP),
                    target.reshape(_BATCH, 1), topsum)
    return loss[0, 0]


# speedup vs baseline: 1.3447x; 1.3447x over previous
import functools

import jax
import jax.numpy as jnp
from jax import lax
from jax.experimental import pallas as pl
from jax.experimental.pallas import tpu as pltpu
from jax.experimental.pallas import tpu_sc as plsc

_VOCAB = 100000
_DIM = 128
_BATCH = 4096
_L = 17
_LSC = 7
_LP = 16
_C = 16
_NC = 2
_NS = 16
_NW = _NC * _NS
_RW = _BATCH // _NW
_NCH = _RW // _C

_WIDE = ((8, 389, 4), (9, 194, 2), (10, 96, 1))


def _sc_body(input_hbm, target_hbm, table_hbm, out_hbm,
             tgt_v, idx8_v, w_v, e_v, dots_v,
             sem_in0, sem_in1, sem_g0, sem_g1, sem_o0, sem_o1):
    sem_in = (sem_in0, sem_in1)
    sem_g = (sem_g0, sem_g1)
    sem_o = (sem_o0, sem_o1)
    wid = lax.axis_index("s") * _NC + lax.axis_index("c")
    lanes = lax.iota(jnp.int32, 16)

    def issue_in(ch):
        p = ch & 1
        base = wid * _RW + ch * _C
        return [
            pltpu.async_copy(target_hbm.at[pl.ds(base, _C)],
                             tgt_v.at[p], sem_in[p]),
            pltpu.async_copy(input_hbm.at[pl.ds(base, _C)],
                             w_v.at[p], sem_in[p]),
        ]

    def idx_and_gather(ch):
        p = ch & 1
        m = tgt_v[p] + _VOCAB
        for kk in range(1, _LSC + 1):
            idx8_v[p, pl.ds((kk - 1) * 16, 16)] = jnp.right_shift(m, kk) - 1
        return [
            pltpu.async_copy(table_hbm.at[idx8_v.at[p, pl.ds(0, _LSC * _C)]],
                             e_v.at[p, pl.ds(0, _LSC * _C)], sem_g[p]),
        ]

    def compute_rows(ch):
        p = ch & 1

        def row_body(b, carry):
            wb = [w_v[p, b, pl.ds(c * 16, 16)] for c in range(8)]
            dots0 = jnp.zeros((16,), jnp.float32)
            for kk in range(_LSC):
                row = kk * _C + b
                acc = e_v[p, row, pl.ds(0, 16)] * wb[0]
                for c in range(1, 8):
                    acc = acc + e_v[p, row, pl.ds(c * 16, 16)] * wb[c]
                for s in (1, 2, 4, 8):
                    acc = acc + acc.at[lanes ^ s].get(
                        mode="promise_in_bounds")
                dots0 = jnp.where(lanes == kk, acc, dots0)
            dots_v[p, b, :] = dots0
            return carry

        lax.fori_loop(0, _C, row_body, 0)

    def issue_out(ch):
        p = ch & 1
        base = wid * _RW + ch * _C
        return pltpu.async_copy(dots_v.at[p],
                                out_hbm.at[pl.ds(base, _C)],
                                sem_o[p])

    in_c = {0: issue_in(0), 1: issue_in(1)}
    for c in in_c[0]:
        c.wait()
    g_c = {0: idx_and_gather(0)}
    out_c = {}
    for ch in range(_NCH):
        if ch + 1 < _NCH:
            for c in in_c[ch + 1]:
                c.wait()
            g_c[ch + 1] = idx_and_gather(ch + 1)
        for c in g_c[ch]:
            c.wait()
        if ch >= 2:
            out_c[ch - 2].wait()
        compute_rows(ch)
        out_c[ch] = issue_out(ch)
        if ch + 2 < _NCH:
            in_c[ch + 2] = issue_in(ch + 2)
    out_c[_NCH - 2].wait()
    out_c[_NCH - 1].wait()


_sc_dots = functools.partial(
    pl.kernel,
    mesh=plsc.VectorSubcoreMesh(core_axis_name="c", subcore_axis_name="s"),
    out_type=jax.ShapeDtypeStruct((_BATCH, _LP), jnp.float32),
    scratch_types=[
        pltpu.VMEM((2, _C), jnp.int32),
        pltpu.VMEM((2, 128), jnp.int32),
        pltpu.VMEM((2, _C, _DIM), jnp.float32),
        pltpu.VMEM((2, _LSC * _C, _DIM), jnp.float32),
        pltpu.VMEM((2, _C, _LP), jnp.float32),
        pltpu.SemaphoreType.DMA,
        pltpu.SemaphoreType.DMA,
        pltpu.SemaphoreType.DMA,
        pltpu.SemaphoreType.DMA,
        pltpu.SemaphoreType.DMA,
        pltpu.SemaphoreType.DMA,
    ],
)(_sc_body)


def _tc_top_body(x_ref, tgt_ref, tab_ref, out_ref):
    x = x_ref[...]
    m = tgt_ref[...] + _VOCAB
    dims = (((1,), (1,)), ((), ()))

    def score(base):
        t = tab_ref[pl.ds(base, 128), :]
        return lax.dot_general(x, t, dims,
                               preferred_element_type=jnp.float32)

    s_lo = score(0)
    kcol = 10 + lax.broadcasted_iota(jnp.int32, (_BATCH, 8), 1)
    mk = jnp.right_shift(m, kcol)
    off_lo = jnp.where((kcol > 10) & (mk >= 1), mk - 1, 0)
    top = jnp.take_along_axis(s_lo, off_lo, axis=1)
    turn = jnp.where((jnp.right_shift(m, kcol - 1) & 1) == 0, 1.0, -1.0)
    coef = jnp.where(mk >= 1, turn, 0.0)
    total = jnp.sum(jax.nn.log_sigmoid(top * coef) *
                    jnp.where(kcol > 10, 1.0, 0.0))
    for k, base, nslices in _WIDE:
        off = jnp.right_shift(m, k) - 1 - base
        sel = None
        for i in range(nslices):
            s_i = score(base + 128 * i)
            c_i = jnp.take_along_axis(
                s_i, jnp.clip(off - 128 * i, 0, 127), axis=1)
            sel = c_i if sel is None else jnp.where(off < 128 * i, sel, c_i)
        turn_k = jnp.where((jnp.right_shift(m, k - 1) & 1) == 0, 1.0, -1.0)
        total = total + jnp.sum(jax.nn.log_sigmoid(sel * turn_k))
    out_ref[0, 0] = total


def _tc_top(input_word, tgt2d, table):
    return pl.pallas_call(
        _tc_top_body,
        grid=(1,),
        in_specs=[
            pl.BlockSpec((_BATCH, _DIM), lambda i: (0, 0)),
            pl.BlockSpec((_BATCH, 1), lambda i: (0, 0)),
            pl.BlockSpec((1024, _DIM), lambda i: (0, 0)),
        ],
        out_shape=jax.ShapeDtypeStruct((1, 1), jnp.float32),
        out_specs=pl.BlockSpec(memory_space=pltpu.SMEM),
    )(input_word, tgt2d, table)


def _tc_loss_body(dots_ref, tgt_ref, top_ref, out_ref):
    m = tgt_ref[...] + _VOCAB
    col = lax.broadcasted_iota(jnp.int32, (_BATCH, _LP), 1)
    turn = jnp.where((jnp.right_shift(m, col) & 1) == 0, 1.0, -1.0)
    x = dots_ref[...] * turn
    ls = jnp.where(col < _LSC, jax.nn.log_sigmoid(x), 0.0)
    out_ref[0, 0] = -(jnp.sum(ls) + top_ref[0, 0]) / _BATCH


def _tc_loss(dots2d, tgt2d, topsum):
    return pl.pallas_call(
        _tc_loss_body,
        in_specs=[
            pl.BlockSpec((_BATCH, _LP), lambda: (0, 0)),
            pl.BlockSpec((_BATCH, 1), lambda: (0, 0)),
            pl.BlockSpec(memory_space=pltpu.SMEM),
        ],
        out_shape=jax.ShapeDtypeStruct((1, 1), jnp.float32),
        out_specs=pl.BlockSpec(memory_space=pltpu.SMEM),
    )(dots2d, tgt2d, topsum)


def kernel(input_word, target, output_matrix):
    dots = _sc_dots(input_word, target, output_matrix)
    topsum = _tc_top(input_word, target.reshape(_BATCH, 1), output_matrix)
    loss = _tc_loss(dots, target.reshape(_BATCH, 1), topsum)
    return loss[0, 0]
